# Initial kernel scaffold; baseline (speedup 1.0000x reference)
import jax
import jax.numpy as jnp
from jax.experimental import pallas as pl


def kernel(user_idx, seq_idx, user_table, seq_table):
    raise NotImplementedError("write your pallas kernel here")


# speedup vs baseline: 20.1034x; 20.1034x over previous
import functools

import jax
import jax.numpy as jnp
from jax import lax
from jax.experimental import pallas as pl
from jax.experimental.pallas import tpu as pltpu
from jax.experimental.pallas import tpu_sc as plsc

B = 1024
F = 26
LH = 50
D = 64
NC, NS = 2, 16
NW = NC * NS
BPW = B // NW
KV = D // 16


def _rsqrt_vec(s_vec):
    i = plsc.bitcast(s_vec, jnp.int32)
    i = 0x5F3759DF - lax.shift_right_logical(i, 1)
    y = plsc.bitcast(i, jnp.float32)
    for _ in range(3):
        y = y * (1.5 - 0.5 * s_vec * y * y)
    return y


def _sc_body(uidx_hbm, sidx_hbm, utab_hbm, stab_hbm, out_hbm,
             uidx_v, sidx_v, acc_a, acc_b, buf0, buf1,
             sem_u, sem0, sem1, sem_oa, sem_ob):
    wid = lax.axis_index("s") * NC + lax.axis_index("c")
    base = wid * BPW
    pltpu.sync_copy(uidx_hbm.at[pl.ds(base, BPW)], uidx_v)
    pltpu.sync_copy(sidx_hbm.at[pl.ds(base, BPW)], sidx_v)

    def do_row(b, acc, sem_o):
        cp_u = pltpu.async_copy(utab_hbm.at[uidx_v.at[b]], acc, sem_u)
        pltpu.async_copy(stab_hbm.at[sidx_v.at[b, 0]], buf0, sem0)
        pltpu.async_copy(stab_hbm.at[sidx_v.at[b, 1]], buf1, sem1)
        cp_u.wait()

        def field_pair(c, sq):
            f0 = 2 * c

            def bag(f, buf, sem):
                pltpu.make_async_copy(stab_hbm.at[sidx_v.at[b, f]], buf, sem).wait()
                v = [acc[f, pl.ds(k * 16, 16)] for k in range(KV)]
                for l in range(LH):
                    for k in range(KV):
                        v[k] = v[k] + buf[l, pl.ds(k * 16, 16)]
                ssq = jnp.zeros((16,), jnp.float32)
                for k in range(KV):
                    acc[f, pl.ds(k * 16, 16)] = v[k]
                    ssq = ssq + v[k] * v[k]

                @pl.when(c < (F // 2) - 1)
                def _():
                    pltpu.async_copy(stab_hbm.at[sidx_v.at[b, f + 2]], buf, sem)

                return ssq

            return sq + bag(f0, buf0, sem0) + bag(f0 + 1, buf1, sem1)

        sq = lax.fori_loop(0, F // 2, field_pair, jnp.zeros((16,), jnp.float32))
        s = jnp.maximum(jnp.sum(sq), 1e-24)
        y = _rsqrt_vec(jnp.full((16,), s, jnp.float32))

        def scale(f, carry):
            for k in range(KV):
                acc[f, pl.ds(k * 16, 16)] = acc[f, pl.ds(k * 16, 16)] * y
            return carry

        lax.fori_loop(0, F, scale, 0)
        pltpu.async_copy(acc, out_hbm.at[base + b], sem_o)

    def pair_step(i, carry):
        @pl.when(i > 0)
        def _():
            pltpu.make_async_copy(out_hbm.at[0], acc_a, sem_oa).wait()
            pltpu.make_async_copy(out_hbm.at[0], acc_b, sem_ob).wait()

        do_row(2 * i, acc_a, sem_oa)
        do_row(2 * i + 1, acc_b, sem_ob)
        return carry

    lax.fori_loop(0, BPW // 2, pair_step, 0)
    pltpu.make_async_copy(out_hbm.at[0], acc_a, sem_oa).wait()
    pltpu.make_async_copy(out_hbm.at[0], acc_b, sem_ob).wait()


@jax.jit
def kernel(user_idx, seq_idx, user_table, seq_table):
    mesh = plsc.VectorSubcoreMesh(core_axis_name="c", subcore_axis_name="s")
    run = pl.kernel(
        _sc_body,
        out_type=jax.ShapeDtypeStruct((B, F, D), jnp.float32),
        mesh=mesh,
        scratch_types=[
            pltpu.VMEM((BPW, F), jnp.int32),
            pltpu.VMEM((BPW, F, LH), jnp.int32),
            pltpu.VMEM((F, D), jnp.float32),
            pltpu.VMEM((F, D), jnp.float32),
            pltpu.VMEM((LH, D), jnp.float32),
            pltpu.VMEM((LH, D), jnp.float32),
            pltpu.SemaphoreType.DMA,
            pltpu.SemaphoreType.DMA,
            pltpu.SemaphoreType.DMA,
            pltpu.SemaphoreType.DMA,
            pltpu.SemaphoreType.DMA,
        ],
        compiler_params=pltpu.CompilerParams(
            use_tc_tiling_on_sc=False, needs_layout_passes=False),
    )
    out = run(user_idx, seq_idx, user_table, seq_table)
    return out.reshape(B, F * D)


# speedup vs baseline: 33.5277x; 1.6678x over previous
import jax
import jax.numpy as jnp
from jax import lax
from jax.experimental import pallas as pl
from jax.experimental.pallas import tpu as pltpu
from jax.experimental.pallas import tpu_sc as plsc

B = 1024
F = 26
LH = 50
D = 64
NC, NS = 2, 16
NW = NC * NS
BPW = B // NW
KV = D // 16
CPR = F // 2
NCHUNK = BPW * CPR
NBUF = 4


def _rsqrt_vec(s_vec):
    i = plsc.bitcast(s_vec, jnp.int32)
    i = 0x5F3759DF - lax.shift_right_logical(i, 1)
    y = plsc.bitcast(i, jnp.float32)
    for _ in range(3):
        y = y * (1.5 - 0.5 * s_vec * y * y)
    return y


def _sc_body(uidx_hbm, sidx_hbm, utab_hbm, stab_hbm, out_hbm,
             uidx_v, sidx_v, ustage, accs, bufs,
             sem_u, sem_g, sem_o):
    wid = lax.axis_index("s") * NC + lax.axis_index("c")
    base = wid * BPW
    pltpu.sync_copy(uidx_hbm.at[pl.ds(base, BPW)], uidx_v)
    pltpu.sync_copy(sidx_hbm.at[pl.ds(base, BPW)], sidx_v)

    cps = [pltpu.async_copy(utab_hbm.at[uidx_v.at[b]], ustage.at[b], sem_u)
           for b in range(BPW)]
    for j in range(NBUF):
        pltpu.async_copy(stab_hbm.at[sidx_v.at[0, j]], bufs.at[j], sem_g.at[j])
    for cp in cps:
        cp.wait()

    def chunk_step(g, sq_in):
        b = g // CPR
        c = g - b * CPR
        slot = lax.rem(g, NBUF)
        par = lax.rem(b, 2)

        @pl.when((c == 0) & (b >= 2))
        def _():
            pltpu.make_async_copy(out_hbm.at[0], accs.at[0], sem_o.at[par]).wait()

        pltpu.make_async_copy(stab_hbm.at[sidx_v.at[b, c]], bufs.at[slot],
                              sem_g.at[slot]).wait()

        sq = jnp.where(c == 0, jnp.zeros((16,), jnp.float32), sq_in)
        for half in range(2):
            f = 2 * c + half
            v = [ustage[b, f, pl.ds(k * 16, 16)] for k in range(KV)]
            for l in range(LH):
                for k in range(KV):
                    v[k] = v[k] + bufs[slot, half * LH + l, pl.ds(k * 16, 16)]
            for k in range(KV):
                accs[par, f, pl.ds(k * 16, 16)] = v[k]
                sq = sq + v[k] * v[k]

        @pl.when(g < NCHUNK - NBUF)
        def _():
            g2 = g + NBUF
            b2 = g2 // CPR
            c2 = g2 - b2 * CPR
            pltpu.async_copy(stab_hbm.at[sidx_v.at[b2, c2]], bufs.at[slot],
                             sem_g.at[slot])

        @pl.when(c == CPR - 1)
        def _():
            s = jnp.maximum(jnp.sum(sq), 1e-24)
            y = _rsqrt_vec(jnp.full((16,), s, jnp.float32))

            def scale(f, carry):
                for k in range(KV):
                    accs[par, f, pl.ds(k * 16, 16)] = (
                        accs[par, f, pl.ds(k * 16, 16)] * y)
                return carry

            lax.fori_loop(0, F, scale, 0)
            pltpu.async_copy(accs.at[par], out_hbm.at[base + b], sem_o.at[par])

        return sq

    lax.fori_loop(0, NCHUNK, chunk_step, jnp.zeros((16,), jnp.float32))
    pltpu.make_async_copy(out_hbm.at[0], accs.at[0], sem_o.at[0]).wait()
    pltpu.make_async_copy(out_hbm.at[0], accs.at[0], sem_o.at[1]).wait()


@jax.jit
def kernel(user_idx, seq_idx, user_table, seq_table):
    mesh = plsc.VectorSubcoreMesh(core_axis_name="c", subcore_axis_name="s")
    run = pl.kernel(
        _sc_body,
        out_type=jax.ShapeDtypeStruct((B, F, D), jnp.float32),
        mesh=mesh,
        scratch_types=[
            pltpu.VMEM((BPW, F), jnp.int32),
            pltpu.VMEM((BPW, CPR, 2 * LH), jnp.int32),
            pltpu.VMEM((BPW, F, D), jnp.float32),
            pltpu.VMEM((2, F, D), jnp.float32),
            pltpu.VMEM((NBUF, 2 * LH, D), jnp.float32),
            pltpu.SemaphoreType.DMA,
            pltpu.SemaphoreType.DMA((NBUF,)),
            pltpu.SemaphoreType.DMA((2,)),
        ],
        compiler_params=pltpu.CompilerParams(
            use_tc_tiling_on_sc=False, needs_layout_passes=False),
    )
    out = run(user_idx, seq_idx.reshape(B, CPR, 2 * LH), user_table, seq_table)
    return out.reshape(B, F * D)
